# Initial kernel scaffold; baseline (speedup 1.0000x reference)
import jax
import jax.numpy as jnp
from jax.experimental import pallas as pl


def kernel(means1, stds1, means4_slope, stds4_slope, means4_sum, stds4_sum, means16_slope, stds16_slope, means16_sum, stds16_sum, norm1, norm4, norm16):
    raise NotImplementedError("write your pallas kernel here")


# speedup vs baseline: 2.7161x; 2.7161x over previous
import functools

import jax
import jax.numpy as jnp
from jax import lax
from jax.experimental import pallas as pl
from jax.experimental.pallas import tpu as pltpu
from jax.experimental.pallas import tpu_sc as plsc

B = 64
T = 4096
NCOL = T // 16
NCHUNK = NCOL // 16
N_STEPS = 20
LR = 0.05

OFF_EX = 0
OFF_SD = 4096
OFF_M4S = 8192
OFF_S4S = 9216
OFF_M4M = 10240
OFF_S4M = 11264
OFF_M16S = 12288
OFF_S16S = 12544
OFF_M16M = 12800
OFF_S16M = 13056
OFF_NORM = 13312
PACK = OFF_NORM + 6 * 16

OUT_EX = 0
OUT_SD = 4096
OPACK = 8192

NC, NS = 2, 16
ROWS_PER_W = B // (NC * NS)


def _sc_body(pk_hbm, out_hbm, pk_v, out_v, sem_in, sem_out):
    wid = lax.axis_index("s") * NC + lax.axis_index("c")

    def run_row(rr):
        b = wid * ROWS_PER_W + rr
        pltpu.async_copy(pk_hbm.at[b], pk_v, sem_in).wait()

        def nconst(k):
            return pk_v[pl.ds(OFF_NORM + 16 * k, 16)]

        n1_0, n1_1 = nconst(0), nconst(1)
        n4_0, n4_1 = nconst(2), nconst(3)
        n16_0, n16_1 = nconst(4), nconst(5)
        s1 = n1_1 + 0.5
        o1 = n1_0
        r4 = 1.0 / (n4_1 + 0.5)
        r16 = 1.0 / (n16_1 + 0.5)
        A16 = s1 * r16 * (1.0 / 16.0)
        C16 = (o1 - n16_0) * r16
        D16 = s1 * r16 * (1.0 / 340.0)
        E16 = 0.0 - n16_0 * r16
        A4 = s1 * r4 * 0.25
        C4 = (o1 - n4_0) * r4
        D4 = s1 * r4 * 0.2
        E4 = 0.0 - n4_0 * r4

        def chunk_body(c, carry):
            base = c * 16
            ex0 = [pk_v[pl.ds(OFF_EX + j * NCOL + base, 16)] for j in range(16)]
            iv1, p1 = [], []
            for j in range(16):
                sd = jnp.abs(pk_v[pl.ds(OFF_SD + j * NCOL + base, 16)]) + 0.5
                out_v[pl.ds(OUT_SD + j * NCOL + base, 16)] = sd
                iv = 1.0 / (sd * sd)
                iv1.append(iv)
                p1.append(ex0[j] * iv)

            def coeff(off_m, off_s, g=0):
                ivx = jnp.abs(pk_v[pl.ds(off_s + g * NCOL + base, 16)]) + 0.5
                ivx = 1.0 / (ivx * ivx)
                px = pk_v[pl.ds(off_m + g * NCOL + base, 16)] * ivx
                return ivx, px

            ivm16, pm16 = coeff(OFF_M16M, OFF_S16M)
            ivs16, ps16 = coeff(OFF_M16S, OFF_S16S)
            lvl4 = [(coeff(OFF_M4M, OFF_S4M, g), coeff(OFF_M4S, OFF_S4S, g))
                    for g in range(4)]

            def step_body(_, exs):
                S16 = exs[0]
                for j in range(1, 16):
                    S16 = S16 + exs[j]
                W16 = (0 - 7.5) * exs[0]
                for j in range(1, 16):
                    W16 = W16 + (j - 7.5) * exs[j]
                ag16 = S16 * A16 + C16
                t16m = (ag16 * ivm16 - pm16) * A16
                sl16 = W16 * D16 + E16
                t16s = (sl16 * ivs16 - ps16) * D16
                t4m, t4s = [], []
                for g in range(4):
                    (ivm4, pm4), (ivs4, ps4) = lvl4[g]
                    S4 = exs[4 * g] + exs[4 * g + 1] + exs[4 * g + 2] + exs[4 * g + 3]
                    W4 = ((0 - 1.5) * exs[4 * g] + (0 - 0.5) * exs[4 * g + 1]
                          + 0.5 * exs[4 * g + 2] + 1.5 * exs[4 * g + 3])
                    ag4 = S4 * A4 + C4
                    t4m.append((ag4 * ivm4 - pm4) * A4)
                    sl4 = W4 * D4 + E4
                    t4s.append((sl4 * ivs4 - ps4) * D4)
                new = []
                for j in range(16):
                    g = j // 4
                    grad = (exs[j] * iv1[j] - p1[j] + t16m + (j - 7.5) * t16s
                            + t4m[g] + (j % 4 - 1.5) * t4s[g])
                    new.append(exs[j] - LR * grad)
                return tuple(new)

            exf = lax.fori_loop(0, N_STEPS, step_body, tuple(ex0))
            for j in range(16):
                out_v[pl.ds(OUT_EX + j * NCOL + base, 16)] = exf[j]
            return carry

        lax.fori_loop(0, NCHUNK, chunk_body, 0)
        pltpu.async_copy(out_v, out_hbm.at[b], sem_out).wait()

    for rr in range(ROWS_PER_W):
        run_row(rr)


@jax.jit
def _run(pk):
    f = pl.kernel(
        _sc_body,
        out_type=jax.ShapeDtypeStruct((B, OPACK), jnp.float32),
        mesh=plsc.VectorSubcoreMesh(
            core_axis_name="c", subcore_axis_name="s",
            num_cores=NC, num_subcores=NS),
        scratch_types=[
            pltpu.VMEM((PACK,), jnp.float32),
            pltpu.VMEM((OPACK,), jnp.float32),
            pltpu.SemaphoreType.DMA,
            pltpu.SemaphoreType.DMA,
        ],
    )
    return f(pk)


def kernel(means1, stds1, means4_slope, stds4_slope, means4_sum, stds4_sum,
           means16_slope, stds16_slope, means16_sum, stds16_sum,
           norm1, norm4, norm16):
    def t16(a):
        return a.reshape(B, NCOL, 16).transpose(0, 2, 1).reshape(B, T)

    def t4(a):
        return a.reshape(B, NCOL, 4).transpose(0, 2, 1).reshape(B, 4 * NCOL)

    normb = jnp.repeat(
        jnp.concatenate([norm1, norm4, norm16], axis=1), 16, axis=1)
    pk = jnp.concatenate(
        [t16(means1), t16(stds1),
         t4(means4_slope), t4(stds4_slope),
         t4(means4_sum), t4(stds4_sum),
         means16_slope, stds16_slope, means16_sum, stds16_sum,
         normb], axis=1)
    out = _run(pk)

    def u16(a):
        return a.reshape(B, 16, NCOL).transpose(0, 2, 1).reshape(B, T)

    ex_final = u16(out[:, OUT_EX:OUT_EX + T])
    all_preds_std = u16(out[:, OUT_SD:OUT_SD + T])
    return ex_final, all_preds_std


# speedup vs baseline: 5.5152x; 2.0306x over previous
import jax
import jax.numpy as jnp
from jax import lax
from jax.experimental import pallas as pl
from jax.experimental.pallas import tpu as pltpu
from jax.experimental.pallas import tpu_sc as plsc

B = 64
T = 4096
NCOL = T // 16
NCHUNK = NCOL // 16
N_STEPS = 20
LR = 0.05

OFF_EX = 0
OFF_SD = 4096
OFF_M4S = 8192
OFF_S4S = 9216
OFF_M4M = 10240
OFF_S4M = 11264
OFF_M16S = 12288
OFF_S16S = 12544
OFF_M16M = 12800
OFF_S16M = 13056
OFF_NORM = 13312
PACK = OFF_NORM + 6 * 16

OUT_EX = 0
OUT_SD = 4096
OPACK = 8192

NC, NS = 2, 16
ROWS_PER_W = B // (NC * NS)


def _compute_row(pk_v, out_v):
    def nconst(k):
        return pk_v[pl.ds(OFF_NORM + 16 * k, 16)]

    n1_0, n1_1 = nconst(0), nconst(1)
    n4_0, n4_1 = nconst(2), nconst(3)
    n16_0, n16_1 = nconst(4), nconst(5)
    s1 = n1_1 + 0.5
    o1 = n1_0
    r4 = 1.0 / (n4_1 + 0.5)
    r16 = 1.0 / (n16_1 + 0.5)
    A16 = s1 * r16 * (1.0 / 16.0)
    C16 = (o1 - n16_0) * r16
    D16 = s1 * r16 * (1.0 / 340.0)
    E16 = 0.0 - n16_0 * r16
    A4 = s1 * r4 * 0.25
    C4 = (o1 - n4_0) * r4
    D4 = s1 * r4 * 0.2
    E4 = 0.0 - n4_0 * r4
    nLRA16 = (0.0 - LR) * A16
    nLRD16 = (0.0 - LR) * D16
    nLRA4 = (0.0 - LR) * A4
    nLRD4 = (0.0 - LR) * D4

    def chunk_body(c, carry):
        base = c * 16
        m1 = [pk_v[pl.ds(OFF_EX + j * NCOL + base, 16)] for j in range(16)]
        d = []
        for j in range(16):
            sd = jnp.abs(pk_v[pl.ds(OFF_SD + j * NCOL + base, 16)]) + 0.5
            out_v[pl.ds(OUT_SD + j * NCOL + base, 16)] = sd
            d.append(1.0 - LR / (sd * sd))

        Sm4 = [m1[4 * g] + m1[4 * g + 1] + m1[4 * g + 2] + m1[4 * g + 3]
               for g in range(4)]
        Sm16 = (Sm4[0] + Sm4[1]) + (Sm4[2] + Sm4[3])
        Wm4 = [1.5 * (m1[4 * g + 3] - m1[4 * g]) + 0.5 * (m1[4 * g + 2] - m1[4 * g + 1])
               for g in range(4)]
        Wm16 = ((Wm4[0] + Wm4[1]) + (Wm4[2] + Wm4[3])
                + (-6.0 * Sm4[0] - 2.0 * Sm4[1] + 2.0 * Sm4[2] + 6.0 * Sm4[3]))

        def coeff(off_m, off_s, g=0):
            sx = jnp.abs(pk_v[pl.ds(off_s + g * NCOL + base, 16)]) + 0.5
            ivx = 1.0 / (sx * sx)
            px = pk_v[pl.ds(off_m + g * NCOL + base, 16)] * ivx
            return ivx, px

        ivm16, pm16 = coeff(OFF_M16M, OFF_S16M)
        P16m = nLRA16 * A16 * ivm16
        Q16m = nLRA16 * ((Sm16 * A16 + C16) * ivm16 - pm16)
        ivs16, ps16 = coeff(OFF_M16S, OFF_S16S)
        P16s = nLRD16 * D16 * ivs16
        Q16s = nLRD16 * ((Wm16 * D16 + E16) * ivs16 - ps16)
        P4m, Q4m, P4s, Q4s = [], [], [], []
        for g in range(4):
            ivm4, pm4 = coeff(OFF_M4M, OFF_S4M, g)
            P4m.append(nLRA4 * A4 * ivm4)
            Q4m.append(nLRA4 * ((Sm4[g] * A4 + C4) * ivm4 - pm4))
            ivs4, ps4 = coeff(OFF_M4S, OFF_S4S, g)
            P4s.append(nLRD4 * D4 * ivs4)
            Q4s.append(nLRD4 * ((Wm4[g] * D4 + E4) * ivs4 - ps4))

        zero = jnp.zeros((16,), jnp.float32)

        def step_body(_, zs):
            S4 = [zs[4 * g] + zs[4 * g + 1] + zs[4 * g + 2] + zs[4 * g + 3]
                  for g in range(4)]
            S16 = (S4[0] + S4[1]) + (S4[2] + S4[3])
            W4 = [1.5 * (zs[4 * g + 3] - zs[4 * g]) + 0.5 * (zs[4 * g + 2] - zs[4 * g + 1])
                  for g in range(4)]
            W16 = ((W4[0] + W4[1]) + (W4[2] + W4[3])
                   + (-6.0 * S4[0] - 2.0 * S4[1] + 2.0 * S4[2] + 6.0 * S4[3]))
            t16m = S16 * P16m + Q16m
            t16s = W16 * P16s + Q16s
            new = []
            for g in range(4):
                t4m = S4[g] * P4m[g] + Q4m[g]
                t4s = W4[g] * P4s[g] + Q4s[g]
                bg = (t16m + t4m) + (4.0 * g - 6.0) * t16s
                sg = t16s + t4s
                for q in range(4):
                    j = 4 * g + q
                    cc = bg + (q - 1.5) * sg
                    new.append(zs[j] * d[j] + cc)
            return tuple(new)

        zf = lax.fori_loop(0, N_STEPS, step_body, (zero,) * 16)
        for j in range(16):
            out_v[pl.ds(OUT_EX + j * NCOL + base, 16)] = zf[j] + m1[j]
        return carry

    lax.fori_loop(0, NCHUNK, chunk_body, 0)


def _sc_body(pk_hbm, out_hbm, pk_v0, pk_v1, out_v0, out_v1,
             sem_i0, sem_i1, sem_o0, sem_o1):
    wid = lax.axis_index("s") * NC + lax.axis_index("c")
    b0 = wid * ROWS_PER_W
    b1 = b0 + 1
    cp0 = pltpu.async_copy(pk_hbm.at[b0], pk_v0, sem_i0)
    cp1 = pltpu.async_copy(pk_hbm.at[b1], pk_v1, sem_i1)
    cp0.wait()
    _compute_row(pk_v0, out_v0)
    w0 = pltpu.async_copy(out_v0, out_hbm.at[b0], sem_o0)
    cp1.wait()
    _compute_row(pk_v1, out_v1)
    w1 = pltpu.async_copy(out_v1, out_hbm.at[b1], sem_o1)
    w0.wait()
    w1.wait()


@jax.jit
def _run(pk):
    f = pl.kernel(
        _sc_body,
        out_type=jax.ShapeDtypeStruct((B, OPACK), jnp.float32),
        mesh=plsc.VectorSubcoreMesh(
            core_axis_name="c", subcore_axis_name="s",
            num_cores=NC, num_subcores=NS),
        scratch_types=[
            pltpu.VMEM((PACK,), jnp.float32),
            pltpu.VMEM((PACK,), jnp.float32),
            pltpu.VMEM((OPACK,), jnp.float32),
            pltpu.VMEM((OPACK,), jnp.float32),
            pltpu.SemaphoreType.DMA,
            pltpu.SemaphoreType.DMA,
            pltpu.SemaphoreType.DMA,
            pltpu.SemaphoreType.DMA,
        ],
    )
    return f(pk)


def kernel(means1, stds1, means4_slope, stds4_slope, means4_sum, stds4_sum,
           means16_slope, stds16_slope, means16_sum, stds16_sum,
           norm1, norm4, norm16):
    def t16(a):
        return a.reshape(B, NCOL, 16).transpose(0, 2, 1).reshape(B, T)

    def t4(a):
        return a.reshape(B, NCOL, 4).transpose(0, 2, 1).reshape(B, 4 * NCOL)

    normb = jnp.repeat(
        jnp.concatenate([norm1, norm4, norm16], axis=1), 16, axis=1)
    pk = jnp.concatenate(
        [t16(means1), t16(stds1),
         t4(means4_slope), t4(stds4_slope),
         t4(means4_sum), t4(stds4_sum),
         means16_slope, stds16_slope, means16_sum, stds16_sum,
         normb], axis=1)
    out = _run(pk)

    def u16(a):
        return a.reshape(B, 16, NCOL).transpose(0, 2, 1).reshape(B, T)

    ex_final = u16(out[:, OUT_EX:OUT_EX + T])
    all_preds_std = u16(out[:, OUT_SD:OUT_SD + T])
    return ex_final, all_preds_std


# speedup vs baseline: 7.8255x; 1.4189x over previous
import jax
import jax.numpy as jnp
from jax import lax
from jax.experimental import pallas as pl
from jax.experimental.pallas import tpu as pltpu
from jax.experimental.pallas import tpu_sc as plsc

B = 64
T = 4096
NCOL = T // 16
NCHUNK = NCOL // 16
N_STEPS = 20
LR = 0.05

NC, NS = 2, 16
ROWS_PER_W = B // (NC * NS)


def _compute_row(mv, sv, m4s_v, s4s_v, m4m_v, s4m_v,
                 m16s_v, s16s_v, m16m_v, s16m_v, nb_v, exo, sdo):
    def nconst(k):
        return nb_v[pl.ds(16 * k, 16)]

    n1_0, n1_1 = nconst(0), nconst(1)
    n4_0, n4_1 = nconst(2), nconst(3)
    n16_0, n16_1 = nconst(4), nconst(5)
    s1 = n1_1 + 0.5
    o1 = n1_0
    r4 = 1.0 / (n4_1 + 0.5)
    r16 = 1.0 / (n16_1 + 0.5)
    A16 = s1 * r16 * (1.0 / 16.0)
    C16 = (o1 - n16_0) * r16
    D16 = s1 * r16 * (1.0 / 340.0)
    E16 = 0.0 - n16_0 * r16
    A4 = s1 * r4 * 0.25
    C4 = (o1 - n4_0) * r4
    D4 = s1 * r4 * 0.2
    E4 = 0.0 - n4_0 * r4
    nLRA16 = (0.0 - LR) * A16
    nLRD16 = (0.0 - LR) * D16
    nLRA4 = (0.0 - LR) * A4
    nLRD4 = (0.0 - LR) * D4

    def chunk_body(c, carry):
        iota = lax.iota(jnp.int32, 16)
        gbase = c * 256 + iota * 16
        g4base = c * 64 + iota * 4
        m1 = [plsc.load_gather(mv, [gbase + j]) for j in range(16)]
        d = []
        for j in range(16):
            sd = jnp.abs(plsc.load_gather(sv, [gbase + j])) + 0.5
            plsc.store_scatter(sdo, [gbase + j], sd)
            d.append(1.0 - LR / (sd * sd))

        Sm4 = [m1[4 * g] + m1[4 * g + 1] + m1[4 * g + 2] + m1[4 * g + 3]
               for g in range(4)]
        Sm16 = (Sm4[0] + Sm4[1]) + (Sm4[2] + Sm4[3])
        Wm4 = [1.5 * (m1[4 * g + 3] - m1[4 * g]) + 0.5 * (m1[4 * g + 2] - m1[4 * g + 1])
               for g in range(4)]
        Wm16 = ((Wm4[0] + Wm4[1]) + (Wm4[2] + Wm4[3])
                + (-6.0 * Sm4[0] - 2.0 * Sm4[1] + 2.0 * Sm4[2] + 6.0 * Sm4[3]))

        def coeff16(ref_m, ref_s):
            sx = jnp.abs(ref_s[pl.ds(c * 16, 16)]) + 0.5
            ivx = 1.0 / (sx * sx)
            return ivx, ref_m[pl.ds(c * 16, 16)] * ivx

        def coeff4(ref_m, ref_s, g):
            sx = jnp.abs(plsc.load_gather(ref_s, [g4base + g])) + 0.5
            ivx = 1.0 / (sx * sx)
            return ivx, plsc.load_gather(ref_m, [g4base + g]) * ivx

        ivm16, pm16 = coeff16(m16m_v, s16m_v)
        P16m = nLRA16 * A16 * ivm16
        Q16m = nLRA16 * ((Sm16 * A16 + C16) * ivm16 - pm16)
        ivs16, ps16 = coeff16(m16s_v, s16s_v)
        P16s = nLRD16 * D16 * ivs16
        Q16s = nLRD16 * ((Wm16 * D16 + E16) * ivs16 - ps16)
        P4m, Q4m, P4s, Q4s = [], [], [], []
        for g in range(4):
            ivm4, pm4 = coeff4(m4m_v, s4m_v, g)
            P4m.append(nLRA4 * A4 * ivm4)
            Q4m.append(nLRA4 * ((Sm4[g] * A4 + C4) * ivm4 - pm4))
            ivs4, ps4 = coeff4(m4s_v, s4s_v, g)
            P4s.append(nLRD4 * D4 * ivs4)
            Q4s.append(nLRD4 * ((Wm4[g] * D4 + E4) * ivs4 - ps4))

        zero = jnp.zeros((16,), jnp.float32)

        def step_body(_, zs):
            S4 = [zs[4 * g] + zs[4 * g + 1] + zs[4 * g + 2] + zs[4 * g + 3]
                  for g in range(4)]
            S16 = (S4[0] + S4[1]) + (S4[2] + S4[3])
            W4 = [1.5 * (zs[4 * g + 3] - zs[4 * g]) + 0.5 * (zs[4 * g + 2] - zs[4 * g + 1])
                  for g in range(4)]
            W16 = ((W4[0] + W4[1]) + (W4[2] + W4[3])
                   + (-6.0 * S4[0] - 2.0 * S4[1] + 2.0 * S4[2] + 6.0 * S4[3]))
            t16m = S16 * P16m + Q16m
            t16s = W16 * P16s + Q16s
            new = []
            for g in range(4):
                t4m = S4[g] * P4m[g] + Q4m[g]
                t4s = W4[g] * P4s[g] + Q4s[g]
                bg = (t16m + t4m) + (4.0 * g - 6.0) * t16s
                sg = t16s + t4s
                for q in range(4):
                    j = 4 * g + q
                    cc = bg + (q - 1.5) * sg
                    new.append(zs[j] * d[j] + cc)
            return tuple(new)

        zf = lax.fori_loop(0, N_STEPS, step_body, (zero,) * 16)
        for j in range(16):
            plsc.store_scatter(exo, [gbase + j], zf[j] + m1[j])
        return carry

    lax.fori_loop(0, NCHUNK, chunk_body, 0)


def _sc_body(means1, stds1, m4s, s4s, m4m, s4m, m16s, s16s, m16m, s16m, normb,
             ex_out, sd_out, bufs0, bufs1, out0, out1,
             sem_i0, sem_i1, sem_o0, sem_o1):
    wid = lax.axis_index("s") * NC + lax.axis_index("c")
    b0 = wid * ROWS_PER_W
    b1 = b0 + 1
    ins = (means1, stds1, m4s, s4s, m4m, s4m, m16s, s16s, m16m, s16m, normb)

    def fetch(b, bufs, sem):
        return [pltpu.async_copy(src.at[b], dst, sem)
                for src, dst in zip(ins, bufs)]

    cps0 = fetch(b0, bufs0, sem_i0)
    cps1 = fetch(b1, bufs1, sem_i1)
    for cp in cps0:
        cp.wait()
    _compute_row(*bufs0, *out0)
    w0 = [pltpu.async_copy(out0[0], ex_out.at[b0], sem_o0),
          pltpu.async_copy(out0[1], sd_out.at[b0], sem_o0)]
    for cp in cps1:
        cp.wait()
    _compute_row(*bufs1, *out1)
    w1 = [pltpu.async_copy(out1[0], ex_out.at[b1], sem_o1),
          pltpu.async_copy(out1[1], sd_out.at[b1], sem_o1)]
    for w in w0 + w1:
        w.wait()


def _row_bufs():
    return [
        pltpu.VMEM((T,), jnp.float32),
        pltpu.VMEM((T,), jnp.float32),
        pltpu.VMEM((T // 4,), jnp.float32),
        pltpu.VMEM((T // 4,), jnp.float32),
        pltpu.VMEM((T // 4,), jnp.float32),
        pltpu.VMEM((T // 4,), jnp.float32),
        pltpu.VMEM((NCOL,), jnp.float32),
        pltpu.VMEM((NCOL,), jnp.float32),
        pltpu.VMEM((NCOL,), jnp.float32),
        pltpu.VMEM((NCOL,), jnp.float32),
        pltpu.VMEM((96,), jnp.float32),
    ]


@jax.jit
def _run(means1, stds1, m4s, s4s, m4m, s4m, m16s, s16s, m16m, s16m, normb):
    f = pl.kernel(
        _sc_body,
        out_type=(jax.ShapeDtypeStruct((B, T), jnp.float32),
                  jax.ShapeDtypeStruct((B, T), jnp.float32)),
        mesh=plsc.VectorSubcoreMesh(
            core_axis_name="c", subcore_axis_name="s",
            num_cores=NC, num_subcores=NS),
        scratch_types=[
            _row_bufs(),
            _row_bufs(),
            [pltpu.VMEM((T,), jnp.float32), pltpu.VMEM((T,), jnp.float32)],
            [pltpu.VMEM((T,), jnp.float32), pltpu.VMEM((T,), jnp.float32)],
            pltpu.SemaphoreType.DMA,
            pltpu.SemaphoreType.DMA,
            pltpu.SemaphoreType.DMA,
            pltpu.SemaphoreType.DMA,
        ],
        compiler_params=pltpu.CompilerParams(needs_layout_passes=False),
    )
    return f(means1, stds1, m4s, s4s, m4m, s4m, m16s, s16s, m16m, s16m, normb)


def kernel(means1, stds1, means4_slope, stds4_slope, means4_sum, stds4_sum,
           means16_slope, stds16_slope, means16_sum, stds16_sum,
           norm1, norm4, norm16):
    normb = jnp.repeat(
        jnp.concatenate([norm1, norm4, norm16], axis=1), 16, axis=1)
    ex_final, all_preds_std = _run(
        means1, stds1, means4_slope, stds4_slope, means4_sum, stds4_sum,
        means16_slope, stds16_slope, means16_sum, stds16_sum, normb)
    return ex_final, all_preds_std


# speedup vs baseline: 7.8332x; 1.0010x over previous
import jax
import jax.numpy as jnp
from jax import lax
from jax.experimental import pallas as pl
from jax.experimental.pallas import tpu as pltpu
from jax.experimental.pallas import tpu_sc as plsc

B = 64
T = 4096
NCOL = T // 16
NCHUNK = NCOL // 16
N_STEPS = 20
LR = 0.05

NC, NS = 2, 16
ROWS_PER_W = B // (NC * NS)


def _compute_row(mv, sv, m4s_v, s4s_v, m4m_v, s4m_v,
                 m16s_v, s16s_v, m16m_v, s16m_v, nb_v, exo, sdo):
    def nconst(k):
        return nb_v[pl.ds(16 * k, 16)]

    n1_0, n1_1 = nconst(0), nconst(1)
    n4_0, n4_1 = nconst(2), nconst(3)
    n16_0, n16_1 = nconst(4), nconst(5)
    s1 = n1_1 + 0.5
    o1 = n1_0
    r4 = 1.0 / (n4_1 + 0.5)
    r16 = 1.0 / (n16_1 + 0.5)
    A16 = s1 * r16 * (1.0 / 16.0)
    C16 = (o1 - n16_0) * r16
    D16 = s1 * r16 * (1.0 / 340.0)
    E16 = 0.0 - n16_0 * r16
    A4 = s1 * r4 * 0.25
    C4 = (o1 - n4_0) * r4
    D4 = s1 * r4 * 0.2
    E4 = 0.0 - n4_0 * r4
    nLRA16 = (0.0 - LR) * A16
    nLRD16 = (0.0 - LR) * D16
    nLRA4 = (0.0 - LR) * A4
    nLRD4 = (0.0 - LR) * D4

    def chunk_body(c, carry):
        iota = lax.iota(jnp.int32, 16)
        gbase = c * 256 + iota * 16
        g4base = c * 64 + iota * 4
        m1 = [plsc.load_gather(mv, [gbase + j]) for j in range(16)]
        d = []
        for j in range(16):
            sd = jnp.abs(plsc.load_gather(sv, [gbase + j])) + 0.5
            plsc.store_scatter(sdo, [gbase + j], sd)
            d.append(1.0 - LR / (sd * sd))

        Sm4 = [m1[4 * g] + m1[4 * g + 1] + m1[4 * g + 2] + m1[4 * g + 3]
               for g in range(4)]
        Sm16 = (Sm4[0] + Sm4[1]) + (Sm4[2] + Sm4[3])
        Wm4 = [1.5 * (m1[4 * g + 3] - m1[4 * g]) + 0.5 * (m1[4 * g + 2] - m1[4 * g + 1])
               for g in range(4)]
        Wm16 = ((Wm4[0] + Wm4[1]) + (Wm4[2] + Wm4[3])
                + (-6.0 * Sm4[0] - 2.0 * Sm4[1] + 2.0 * Sm4[2] + 6.0 * Sm4[3]))

        def coeff16(ref_m, ref_s):
            sx = jnp.abs(ref_s[pl.ds(c * 16, 16)]) + 0.5
            ivx = 1.0 / (sx * sx)
            return ivx, ref_m[pl.ds(c * 16, 16)] * ivx

        def coeff4(ref_m, ref_s, g):
            sx = jnp.abs(plsc.load_gather(ref_s, [g4base + g])) + 0.5
            ivx = 1.0 / (sx * sx)
            return ivx, plsc.load_gather(ref_m, [g4base + g]) * ivx

        ivm16, pm16 = coeff16(m16m_v, s16m_v)
        P16m = nLRA16 * A16 * ivm16
        Q16m = nLRA16 * ((Sm16 * A16 + C16) * ivm16 - pm16)
        ivs16, ps16 = coeff16(m16s_v, s16s_v)
        P16s = nLRD16 * D16 * ivs16
        Q16s = nLRD16 * ((Wm16 * D16 + E16) * ivs16 - ps16)
        P4m, P4s, QB, QS = [], [], [], []
        for g in range(4):
            ivm4, pm4 = coeff4(m4m_v, s4m_v, g)
            P4m.append(nLRA4 * A4 * ivm4)
            q4m = nLRA4 * ((Sm4[g] * A4 + C4) * ivm4 - pm4)
            ivs4, ps4 = coeff4(m4s_v, s4s_v, g)
            P4s.append(nLRD4 * D4 * ivs4)
            q4s = nLRD4 * ((Wm4[g] * D4 + E4) * ivs4 - ps4)
            QB.append((Q16m + q4m) + (4.0 * g - 6.0) * Q16s)
            QS.append(Q16s + q4s)

        zero = jnp.zeros((16,), jnp.float32)

        def step(zs):
            S4 = [zs[4 * g] + zs[4 * g + 1] + zs[4 * g + 2] + zs[4 * g + 3]
                  for g in range(4)]
            S16 = (S4[0] + S4[1]) + (S4[2] + S4[3])
            W4 = [1.5 * (zs[4 * g + 3] - zs[4 * g]) + 0.5 * (zs[4 * g + 2] - zs[4 * g + 1])
                  for g in range(4)]
            W16 = ((W4[0] + W4[1]) + (W4[2] + W4[3])
                   + (-6.0 * S4[0] - 2.0 * S4[1] + 2.0 * S4[2] + 6.0 * S4[3]))
            u = S16 * P16m
            v = W16 * P16s
            new = []
            for g in range(4):
                bg = ((S4[g] * P4m[g] + QB[g]) + u) + (4.0 * g - 6.0) * v
                sg = (W4[g] * P4s[g] + QS[g]) + v
                for q in range(4):
                    j = 4 * g + q
                    cc = bg + (q - 1.5) * sg
                    new.append(zs[j] * d[j] + cc)
            return tuple(new)

        def step_body(_, zs):
            return step(step(zs))

        zf = lax.fori_loop(0, N_STEPS // 2, step_body, (zero,) * 16)
        for j in range(16):
            plsc.store_scatter(exo, [gbase + j], zf[j] + m1[j])
        return carry

    lax.fori_loop(0, NCHUNK, chunk_body, 0)


def _sc_body(means1, stds1, m4s, s4s, m4m, s4m, m16s, s16s, m16m, s16m, normb,
             ex_out, sd_out, bufs0, bufs1, out0, out1,
             sem_i0, sem_i1, sem_o0, sem_o1):
    wid = lax.axis_index("s") * NC + lax.axis_index("c")
    b0 = wid * ROWS_PER_W
    b1 = b0 + 1
    ins = (means1, stds1, m4s, s4s, m4m, s4m, m16s, s16s, m16m, s16m, normb)

    def fetch(b, bufs, sem):
        return [pltpu.async_copy(src.at[b], dst, sem)
                for src, dst in zip(ins, bufs)]

    cps0 = fetch(b0, bufs0, sem_i0)
    cps1 = fetch(b1, bufs1, sem_i1)
    for cp in cps0:
        cp.wait()
    _compute_row(*bufs0, *out0)
    w0 = [pltpu.async_copy(out0[0], ex_out.at[b0], sem_o0),
          pltpu.async_copy(out0[1], sd_out.at[b0], sem_o0)]
    for cp in cps1:
        cp.wait()
    _compute_row(*bufs1, *out1)
    w1 = [pltpu.async_copy(out1[0], ex_out.at[b1], sem_o1),
          pltpu.async_copy(out1[1], sd_out.at[b1], sem_o1)]
    for w in w0 + w1:
        w.wait()


def _row_bufs():
    return [
        pltpu.VMEM((T,), jnp.float32),
        pltpu.VMEM((T,), jnp.float32),
        pltpu.VMEM((T // 4,), jnp.float32),
        pltpu.VMEM((T // 4,), jnp.float32),
        pltpu.VMEM((T // 4,), jnp.float32),
        pltpu.VMEM((T // 4,), jnp.float32),
        pltpu.VMEM((NCOL,), jnp.float32),
        pltpu.VMEM((NCOL,), jnp.float32),
        pltpu.VMEM((NCOL,), jnp.float32),
        pltpu.VMEM((NCOL,), jnp.float32),
        pltpu.VMEM((96,), jnp.float32),
    ]


@jax.jit
def _run(means1, stds1, m4s, s4s, m4m, s4m, m16s, s16s, m16m, s16m, normb):
    f = pl.kernel(
        _sc_body,
        out_type=(jax.ShapeDtypeStruct((B, T), jnp.float32),
                  jax.ShapeDtypeStruct((B, T), jnp.float32)),
        mesh=plsc.VectorSubcoreMesh(
            core_axis_name="c", subcore_axis_name="s",
            num_cores=NC, num_subcores=NS),
        scratch_types=[
            _row_bufs(),
            _row_bufs(),
            [pltpu.VMEM((T,), jnp.float32), pltpu.VMEM((T,), jnp.float32)],
            [pltpu.VMEM((T,), jnp.float32), pltpu.VMEM((T,), jnp.float32)],
            pltpu.SemaphoreType.DMA,
            pltpu.SemaphoreType.DMA,
            pltpu.SemaphoreType.DMA,
            pltpu.SemaphoreType.DMA,
        ],
        compiler_params=pltpu.CompilerParams(needs_layout_passes=False),
    )
    return f(means1, stds1, m4s, s4s, m4m, s4m, m16s, s16s, m16m, s16m, normb)


def kernel(means1, stds1, means4_slope, stds4_slope, means4_sum, stds4_sum,
           means16_slope, stds16_slope, means16_sum, stds16_sum,
           norm1, norm4, norm16):
    normb = jnp.repeat(
        jnp.concatenate([norm1, norm4, norm16], axis=1), 16, axis=1)
    ex_final, all_preds_std = _run(
        means1, stds1, means4_slope, stds4_slope, means4_sum, stds4_sum,
        means16_slope, stds16_slope, means16_sum, stds16_sum, normb)
    return ex_final, all_preds_std
